# baseline (speedup 1.0000x reference)
import functools

import jax
import jax.numpy as jnp
from jax import lax
from jax.experimental import pallas as pl
from jax.experimental.pallas import tpu as pltpu
from jax.experimental.pallas import tpu_sc as plsc

B = 16384
D = 16
NC = 2
NS = 16
NW = NC * NS
BPW = B // NW
LANES = 16
RPG = 128 // D
CH = 128
NCH = BPW // CH


def _body(uidx_hbm, midx_hbm, ub_hbm, mb_hbm, ue_hbm, me_hbm, out_hbm,
          uidx_v, midx_v, uq_v, mq_v, ue_c0, me_c0, ue_c1, me_c1,
          ub_v, mb_v, out_v, sem_a0, sem_a1, sem_b):
    wid = lax.axis_index("s") * NC + lax.axis_index("c")
    base = wid * BPW

    pltpu.sync_copy(uidx_hbm.at[pl.ds(base, BPW)], uidx_v)
    pltpu.sync_copy(midx_hbm.at[pl.ds(base, BPW)], midx_v)

    for j in range(BPW // LANES):
        sl = pl.ds(j * LANES, LANES)
        uq_v[sl] = uidx_v[sl] >> 3
        mq_v[sl] = midx_v[sl] >> 3

    bias_copies = []
    for j in range(NCH):
        sl = pl.ds(j * CH, CH)
        bias_copies.append(
            pltpu.async_copy(ub_hbm.at[uidx_v.at[sl]], ub_v.at[sl], sem_b))
        bias_copies.append(
            pltpu.async_copy(mb_hbm.at[midx_v.at[sl]], mb_v.at[sl], sem_b))

    emb_bufs = ((ue_c0, me_c0), (ue_c1, me_c1))
    emb_sems = (sem_a0, sem_a1)

    def fire(p):
        ue_c, me_c = emb_bufs[p % 2]
        sem = emb_sems[p % 2]
        sl = pl.ds(p * CH, CH)
        return (pltpu.async_copy(ue_hbm.at[uq_v.at[sl]], ue_c, sem),
                pltpu.async_copy(me_hbm.at[mq_v.at[sl]], me_c, sem))

    lanes = lax.iota(jnp.int32, LANES)

    def permute(v, idx):
        return lax.gather(
            v, idx[:, None],
            lax.GatherDimensionNumbers(
                offset_dims=(), collapsed_slice_dims=(0,),
                start_index_map=(0,)),
            (1,), mode=lax.GatherScatterMode.PROMISE_IN_BOUNDS)

    def make_blk_body(p, ue_c, me_c):
        def blk_body(lblk, _):
            sl16 = pl.ds(p * CH + lblk * LANES, LANES)
            su_vec = (uidx_v[sl16] & 7) * D
            sm_vec = (midx_v[sl16] & 7) * D
            acc = jnp.zeros((LANES,), jnp.float32)
            for i in range(LANES):
                rl = lblk * LANES + i
                pr = (ue_c[rl, pl.ds(su_vec[i], D)]
                      * me_c[rl, pl.ds(sm_vec[i], D)])
                for s in (1, 2, 4, 8):
                    pr = pr + permute(pr, lanes ^ s)
                acc = jnp.where(lanes == i, pr, acc)
            out_v[sl16] = acc + ub_v[sl16] + mb_v[sl16]
            return _
        return blk_body

    descs = {0: fire(0)}
    for p in range(NCH):
        if p + 1 < NCH:
            descs[p + 1] = fire(p + 1)
        if p == 0:
            for c in bias_copies:
                c.wait()
        d0, d1 = descs.pop(p)
        d0.wait()
        d1.wait()
        ue_c, me_c = emb_bufs[p % 2]
        lax.fori_loop(0, CH // LANES, make_blk_body(p, ue_c, me_c), 0)

    pltpu.sync_copy(out_v, out_hbm.at[pl.ds(base, BPW)])


@jax.jit
def _run(uidx, midx, ub, mb, ue, me):
    mesh = plsc.VectorSubcoreMesh(
        core_axis_name="c", subcore_axis_name="s",
        num_cores=NC, num_subcores=NS)
    f = pl.kernel(
        _body,
        out_type=jax.ShapeDtypeStruct((B,), jnp.float32),
        mesh=mesh,
        scratch_types=[
            pltpu.VMEM((BPW,), jnp.int32),
            pltpu.VMEM((BPW,), jnp.int32),
            pltpu.VMEM((BPW,), jnp.int32),
            pltpu.VMEM((BPW,), jnp.int32),
            pltpu.VMEM((CH, 128), jnp.float32),
            pltpu.VMEM((CH, 128), jnp.float32),
            pltpu.VMEM((CH, 128), jnp.float32),
            pltpu.VMEM((CH, 128), jnp.float32),
            pltpu.VMEM((BPW,), jnp.float32),
            pltpu.VMEM((BPW,), jnp.float32),
            pltpu.VMEM((BPW,), jnp.float32),
            pltpu.SemaphoreType.DMA,
            pltpu.SemaphoreType.DMA,
            pltpu.SemaphoreType.DMA,
        ],
    )
    return f(uidx, midx, ub, mb, ue, me)


def kernel(user_id, movie_id, user_bias_table, movie_bias_table,
           user_emb_table, movie_emb_table):
    out = _run(user_id.astype(jnp.int32).reshape(B),
               movie_id.astype(jnp.int32).reshape(B),
               user_bias_table.reshape(-1), movie_bias_table.reshape(-1),
               user_emb_table.reshape(-1, 128), movie_emb_table.reshape(-1, 128))
    return out.reshape(B, 1)
